# baseline (speedup 1.0000x reference)
import functools

import jax
import jax.numpy as jnp
from jax import lax
from jax.experimental import pallas as pl
from jax.experimental.pallas import tpu as pltpu
from jax.experimental.pallas import tpu_sc as plsc

VOCAB = 100000
DIM = 64
NVALS = 81920

NC = 2
NS = 16
NW = NC * NS

B_PER_W = NVALS // NW
CHUNK = 320
CPW = B_PER_W // CHUNK
NBUF = 4


@functools.partial(
    pl.kernel,
    out_type=(
        jax.ShapeDtypeStruct((NVALS, DIM), jnp.float32),
        jax.ShapeDtypeStruct((NVALS, DIM), jnp.float32),
    ),
    mesh=plsc.VectorSubcoreMesh(core_axis_name="c", subcore_axis_name="s"),
    compiler_params=pltpu.CompilerParams(use_tc_tiling_on_sc=False),
    scratch_types=[
        pltpu.VMEM((B_PER_W,), jnp.int32),
        pltpu.VMEM((B_PER_W,), jnp.int32),
        pltpu.VMEM((NBUF, CHUNK, DIM), jnp.float32),
        [pltpu.SemaphoreType.DMA] * NBUF,
        [pltpu.SemaphoreType.DMA] * NBUF,
    ],
)
def _lookup2(v1_hbm, v2_hbm, t1_hbm, t2_hbm, o1_hbm, o2_hbm,
             idx1_v, idx2_v, bufs, gsems, ssems):
    wid = lax.axis_index("s") * NC + lax.axis_index("c")
    base = wid * B_PER_W

    pltpu.sync_copy(v1_hbm.at[pl.ds(base, B_PER_W)], idx1_v)
    pltpu.sync_copy(v2_hbm.at[pl.ds(base, B_PER_W)], idx2_v)

    for tbl, idx_v, out in ((t1_hbm, idx1_v, o1_hbm), (t2_hbm, idx2_v, o2_hbm)):

        def gather(g, b):
            return pltpu.make_async_copy(
                tbl.at[idx_v.at[pl.ds(g * CHUNK, CHUNK)]], bufs.at[b],
                gsems[b])

        def scatter(g, b):
            return pltpu.make_async_copy(
                bufs.at[b], out.at[pl.ds(base + g * CHUNK, CHUNK)], ssems[b])

        for b in range(NBUF - 1):
            gather(b, b).start()

        @pl.loop(0, CPW, step=NBUF)
        def _(g0):
            for b in range(NBUF):
                g = g0 + b
                gather(g, b).wait()
                scatter(g, b).start()
                h = g + NBUF - 1
                bh = (b + NBUF - 1) % NBUF

                @pl.when(h < CPW)
                def _():
                    @pl.when(h >= NBUF)
                    def _():
                        scatter(h - NBUF, bh).wait()
                    gather(h, bh).start()

        for g in range(CPW - NBUF, CPW):
            scatter(g, g % NBUF).wait()


def kernel(values_f1, values_f2, table_t1, table_t2):
    return _lookup2(values_f1, values_f2, table_t1, table_t2)


# speedup vs baseline: 1.2303x; 1.2303x over previous
import functools

import jax
import jax.numpy as jnp
from jax import lax
from jax.experimental import pallas as pl
from jax.experimental.pallas import tpu as pltpu
from jax.experimental.pallas import tpu_sc as plsc

VOCAB = 100000
DIM = 64
NVALS = 81920

NS = 16

B_PER_W = NVALS // NS
CHUNK = 640
CPW = B_PER_W // CHUNK
NBUF = 3


@functools.partial(
    pl.kernel,
    out_type=jax.ShapeDtypeStruct((NVALS, 2 * DIM), jnp.float32),
    mesh=plsc.VectorSubcoreMesh(
        core_axis_name="c", subcore_axis_name="s", num_cores=1),
    compiler_params=pltpu.CompilerParams(use_tc_tiling_on_sc=False),
    scratch_types=[
        pltpu.VMEM((B_PER_W,), jnp.int32),
        pltpu.VMEM((NBUF, CHUNK, DIM), jnp.float32),
        [pltpu.SemaphoreType.DMA] * NBUF,
        [pltpu.SemaphoreType.DMA] * NBUF,
    ],
)
def _lookup1(vals_hbm, table_hbm, out_hbm, idx_v, bufs, gsems, ssems):
    wid = lax.axis_index("s")
    base = wid * B_PER_W

    pltpu.sync_copy(vals_hbm.at[pl.ds(base, B_PER_W)], idx_v)

    def fire_gather(i):
        return pltpu.async_copy(
            table_hbm.at[idx_v.at[pl.ds(i * CHUNK, CHUNK)]],
            bufs.at[i % NBUF], gsems[i % NBUF])

    def fire_scatter(i):
        return pltpu.async_copy(
            bufs.at[i % NBUF],
            out_hbm.at[pl.ds(base + i * CHUNK, CHUNK), pl.ds(0, DIM)],
            ssems[i % NBUF])

    gh = [None] * CPW
    sh = [None] * CPW
    gh[0] = fire_gather(0)
    gh[1] = fire_gather(1)
    for i in range(CPW):
        gh[i].wait()
        sh[i] = fire_scatter(i)
        if i + 2 < CPW:
            if i >= 1:
                sh[i - 1].wait()
            gh[i + 2] = fire_gather(i + 2)
    for i in range(CPW - NBUF, CPW):
        sh[i].wait()


def kernel(values_f1, values_f2, table_t1, table_t2):
    o1 = _lookup1(values_f1, table_t1)
    o2 = _lookup1(values_f2, table_t2)
    return (o1[:, :DIM], o2[:, :DIM])


# speedup vs baseline: 1.2351x; 1.0039x over previous
import functools

import jax
import jax.numpy as jnp
from jax import lax
from jax.experimental import pallas as pl
from jax.experimental.pallas import tpu as pltpu
from jax.experimental.pallas import tpu_sc as plsc

VOCAB = 100000
DIM = 64
NVALS = 81920

NS = 16

B_PER_W = NVALS // NS
CHUNK = 512
CPW = B_PER_W // CHUNK
NBUF = 3


@functools.partial(
    pl.kernel,
    out_type=jax.ShapeDtypeStruct((NVALS, 2 * DIM), jnp.float32),
    mesh=plsc.VectorSubcoreMesh(
        core_axis_name="c", subcore_axis_name="s", num_cores=1),
    compiler_params=pltpu.CompilerParams(use_tc_tiling_on_sc=False),
    scratch_types=[
        pltpu.VMEM((B_PER_W,), jnp.int32),
        pltpu.VMEM((NBUF, CHUNK, DIM), jnp.float32),
        [pltpu.SemaphoreType.DMA] * NBUF,
        [pltpu.SemaphoreType.DMA] * NBUF,
    ],
)
def _lookup1(vals_hbm, table_hbm, out_hbm, idx_v, bufs, gsems, ssems):
    wid = lax.axis_index("s")
    base = wid * B_PER_W

    pltpu.sync_copy(vals_hbm.at[pl.ds(base, B_PER_W)], idx_v)

    def fire_gather(i):
        return pltpu.async_copy(
            table_hbm.at[idx_v.at[pl.ds(i * CHUNK, CHUNK)]],
            bufs.at[i % NBUF], gsems[i % NBUF])

    def fire_scatter(i):
        return pltpu.async_copy(
            bufs.at[i % NBUF],
            out_hbm.at[pl.ds(base + i * CHUNK, CHUNK), pl.ds(0, DIM)],
            ssems[i % NBUF])

    gh = [None] * CPW
    sh = [None] * CPW
    gh[0] = fire_gather(0)
    gh[1] = fire_gather(1)
    for i in range(CPW):
        gh[i].wait()
        sh[i] = fire_scatter(i)
        if i + 2 < CPW:
            if i >= 1:
                sh[i - 1].wait()
            gh[i + 2] = fire_gather(i + 2)
    for i in range(CPW - NBUF, CPW):
        sh[i].wait()


def kernel(values_f1, values_f2, table_t1, table_t2):
    o1 = _lookup1(values_f1, table_t1)
    o2 = _lookup1(values_f2, table_t2)
    return (o1[:, :DIM], o2[:, :DIM])


# speedup vs baseline: 1.2416x; 1.0053x over previous
import functools

import jax
import jax.numpy as jnp
from jax import lax
from jax.experimental import pallas as pl
from jax.experimental.pallas import tpu as pltpu
from jax.experimental.pallas import tpu_sc as plsc

VOCAB = 100000
DIM = 64
NVALS = 81920

NS = 16

B_PER_W = NVALS // NS
CHUNK = 320
CPW = B_PER_W // CHUNK
NBUF = 4


@functools.partial(
    pl.kernel,
    out_type=jax.ShapeDtypeStruct((NVALS, 2 * DIM), jnp.float32),
    mesh=plsc.VectorSubcoreMesh(
        core_axis_name="c", subcore_axis_name="s", num_cores=1),
    compiler_params=pltpu.CompilerParams(use_tc_tiling_on_sc=False),
    scratch_types=[
        pltpu.VMEM((B_PER_W,), jnp.int32),
        pltpu.VMEM((NBUF, CHUNK, DIM), jnp.float32),
        [pltpu.SemaphoreType.DMA] * NBUF,
        [pltpu.SemaphoreType.DMA] * NBUF,
    ],
)
def _lookup1(vals_hbm, table_hbm, out_hbm, idx_v, bufs, gsems, ssems):
    wid = lax.axis_index("s")
    base = wid * B_PER_W

    pltpu.sync_copy(vals_hbm.at[pl.ds(base, B_PER_W)], idx_v)

    def fire_gather(i):
        return pltpu.async_copy(
            table_hbm.at[idx_v.at[pl.ds(i * CHUNK, CHUNK)]],
            bufs.at[i % NBUF], gsems[i % NBUF])

    def fire_scatter(i):
        return pltpu.async_copy(
            bufs.at[i % NBUF],
            out_hbm.at[pl.ds(base + i * CHUNK, CHUNK), pl.ds(0, DIM)],
            ssems[i % NBUF])

    gh = [None] * CPW
    sh = [None] * CPW
    for i in range(NBUF - 1):
        gh[i] = fire_gather(i)
    for i in range(CPW):
        gh[i].wait()
        sh[i] = fire_scatter(i)
        if i + NBUF - 1 < CPW:
            if i >= 1:
                sh[i - 1].wait()
            gh[i + NBUF - 1] = fire_gather(i + NBUF - 1)
    for i in range(CPW - NBUF, CPW):
        sh[i].wait()


def kernel(values_f1, values_f2, table_t1, table_t2):
    o1 = _lookup1(values_f1, table_t1)
    o2 = _lookup1(values_f2, table_t2)
    return (o1[:, :DIM], o2[:, :DIM])


# speedup vs baseline: 1.2476x; 1.0048x over previous
import functools

import jax
import jax.numpy as jnp
from jax import lax
from jax.experimental import pallas as pl
from jax.experimental.pallas import tpu as pltpu
from jax.experimental.pallas import tpu_sc as plsc

VOCAB = 100000
DIM = 64
NVALS = 81920

NS = 16

B_PER_W = NVALS // NS
CHUNK = 256
CPW = B_PER_W // CHUNK
NBUF = 5


@functools.partial(
    pl.kernel,
    out_type=jax.ShapeDtypeStruct((NVALS, 2 * DIM), jnp.float32),
    mesh=plsc.VectorSubcoreMesh(
        core_axis_name="c", subcore_axis_name="s", num_cores=1),
    compiler_params=pltpu.CompilerParams(use_tc_tiling_on_sc=False),
    scratch_types=[
        pltpu.VMEM((B_PER_W,), jnp.int32),
        pltpu.VMEM((NBUF, CHUNK, DIM), jnp.float32),
        [pltpu.SemaphoreType.DMA] * NBUF,
        [pltpu.SemaphoreType.DMA] * NBUF,
    ],
)
def _lookup1(vals_hbm, table_hbm, out_hbm, idx_v, bufs, gsems, ssems):
    wid = lax.axis_index("s")
    base = wid * B_PER_W

    pltpu.sync_copy(vals_hbm.at[pl.ds(base, B_PER_W)], idx_v)

    def fire_gather(i):
        return pltpu.async_copy(
            table_hbm.at[idx_v.at[pl.ds(i * CHUNK, CHUNK)]],
            bufs.at[i % NBUF], gsems[i % NBUF])

    def fire_scatter(i):
        return pltpu.async_copy(
            bufs.at[i % NBUF],
            out_hbm.at[pl.ds(base + i * CHUNK, CHUNK), pl.ds(0, DIM)],
            ssems[i % NBUF])

    gh = [None] * CPW
    sh = [None] * CPW
    for i in range(NBUF - 1):
        gh[i] = fire_gather(i)
    for i in range(CPW):
        gh[i].wait()
        sh[i] = fire_scatter(i)
        if i + NBUF - 1 < CPW:
            if i >= 1:
                sh[i - 1].wait()
            gh[i + NBUF - 1] = fire_gather(i + NBUF - 1)
    for i in range(CPW - NBUF, CPW):
        sh[i].wait()


def kernel(values_f1, values_f2, table_t1, table_t2):
    o1 = _lookup1(values_f1, table_t1)
    o2 = _lookup1(values_f2, table_t2)
    return (o1[:, :DIM], o2[:, :DIM])


# speedup vs baseline: 1.2503x; 1.0021x over previous
import functools

import jax
import jax.numpy as jnp
from jax import lax
from jax.experimental import pallas as pl
from jax.experimental.pallas import tpu as pltpu
from jax.experimental.pallas import tpu_sc as plsc

VOCAB = 100000
DIM = 64
NVALS = 81920

NS = 16

B_PER_W = NVALS // NS
CHUNK = 128
CPW = B_PER_W // CHUNK
NBUF = 8


@functools.partial(
    pl.kernel,
    out_type=jax.ShapeDtypeStruct((NVALS, 2 * DIM), jnp.float32),
    mesh=plsc.VectorSubcoreMesh(
        core_axis_name="c", subcore_axis_name="s", num_cores=1),
    compiler_params=pltpu.CompilerParams(use_tc_tiling_on_sc=False),
    scratch_types=[
        pltpu.VMEM((B_PER_W,), jnp.int32),
        pltpu.VMEM((NBUF, CHUNK, DIM), jnp.float32),
        [pltpu.SemaphoreType.DMA] * NBUF,
        [pltpu.SemaphoreType.DMA] * NBUF,
    ],
)
def _lookup1(vals_hbm, table_hbm, out_hbm, idx_v, bufs, gsems, ssems):
    wid = lax.axis_index("s")
    base = wid * B_PER_W

    pltpu.sync_copy(vals_hbm.at[pl.ds(base, B_PER_W)], idx_v)

    def fire_gather(i):
        return pltpu.async_copy(
            table_hbm.at[idx_v.at[pl.ds(i * CHUNK, CHUNK)]],
            bufs.at[i % NBUF], gsems[i % NBUF])

    def fire_scatter(i):
        return pltpu.async_copy(
            bufs.at[i % NBUF],
            out_hbm.at[pl.ds(base + i * CHUNK, CHUNK), pl.ds(0, DIM)],
            ssems[i % NBUF])

    gh = [None] * CPW
    sh = [None] * CPW
    for i in range(NBUF - 1):
        gh[i] = fire_gather(i)
    for i in range(CPW):
        gh[i].wait()
        sh[i] = fire_scatter(i)
        if i + NBUF - 1 < CPW:
            if i >= 1:
                sh[i - 1].wait()
            gh[i + NBUF - 1] = fire_gather(i + NBUF - 1)
    for i in range(CPW - NBUF, CPW):
        sh[i].wait()


def kernel(values_f1, values_f2, table_t1, table_t2):
    o1 = _lookup1(values_f1, table_t1)
    o2 = _lookup1(values_f2, table_t2)
    return (o1[:, :DIM], o2[:, :DIM])


# speedup vs baseline: 1.2507x; 1.0003x over previous
import functools

import jax
import jax.numpy as jnp
from jax import lax
from jax.experimental import pallas as pl
from jax.experimental.pallas import tpu as pltpu
from jax.experimental.pallas import tpu_sc as plsc

VOCAB = 100000
DIM = 64
NVALS = 81920

NS = 16

B_PER_W = NVALS // NS
CHUNK = 64
CPW = B_PER_W // CHUNK
NBUF = 12


@functools.partial(
    pl.kernel,
    out_type=jax.ShapeDtypeStruct((NVALS, 2 * DIM), jnp.float32),
    mesh=plsc.VectorSubcoreMesh(
        core_axis_name="c", subcore_axis_name="s", num_cores=1),
    compiler_params=pltpu.CompilerParams(use_tc_tiling_on_sc=False),
    scratch_types=[
        pltpu.VMEM((B_PER_W,), jnp.int32),
        pltpu.VMEM((NBUF, CHUNK, DIM), jnp.float32),
        [pltpu.SemaphoreType.DMA] * NBUF,
        [pltpu.SemaphoreType.DMA] * NBUF,
    ],
)
def _lookup1(vals_hbm, table_hbm, out_hbm, idx_v, bufs, gsems, ssems):
    wid = lax.axis_index("s")
    base = wid * B_PER_W

    pltpu.sync_copy(vals_hbm.at[pl.ds(base, B_PER_W)], idx_v)

    def fire_gather(i):
        return pltpu.async_copy(
            table_hbm.at[idx_v.at[pl.ds(i * CHUNK, CHUNK)]],
            bufs.at[i % NBUF], gsems[i % NBUF])

    def fire_scatter(i):
        return pltpu.async_copy(
            bufs.at[i % NBUF],
            out_hbm.at[pl.ds(base + i * CHUNK, CHUNK), pl.ds(0, DIM)],
            ssems[i % NBUF])

    gh = [None] * CPW
    sh = [None] * CPW
    for i in range(NBUF - 1):
        gh[i] = fire_gather(i)
    for i in range(CPW):
        gh[i].wait()
        sh[i] = fire_scatter(i)
        if i + NBUF - 1 < CPW:
            if i >= 1:
                sh[i - 1].wait()
            gh[i + NBUF - 1] = fire_gather(i + NBUF - 1)
    for i in range(CPW - NBUF, CPW):
        sh[i].wait()


def kernel(values_f1, values_f2, table_t1, table_t2):
    o1 = _lookup1(values_f1, table_t1)
    o2 = _lookup1(values_f2, table_t2)
    return (o1[:, :DIM], o2[:, :DIM])
